# baseline (speedup 1.0000x reference)
import functools

import jax
import jax.numpy as jnp
from jax import lax
from jax.experimental import pallas as pl
from jax.experimental.pallas import tpu as pltpu
from jax.experimental.pallas import tpu_sc as plsc

N = 10000
E = 320000
D = 128
H = 16
C = 7

NC = 2
NS = 16
NW = NC * NS
CHUNK = 128
ER = E // CHUNK
KW = 78
XW = ER - NW * KW
NBUF = 6
NP = 10240
RPT = NP // NS

_mesh = plsc.VectorSubcoreMesh(core_axis_name="c", subcore_axis_name="s")
_sc_params = pltpu.CompilerParams(use_tc_tiling_on_sc=False)


def _sc_degree(dst2):
    @functools.partial(
        pl.kernel,
        out_type=jax.ShapeDtypeStruct((NC, NP), jnp.float32),
        mesh=_mesh,
        compiler_params=_sc_params,
        scratch_types=[
            pltpu.VMEM_SHARED((NP,), jnp.float32),
            pltpu.VMEM((KW + 1, CHUNK), jnp.int32),
            pltpu.VMEM((CHUNK,), jnp.float32),
            pltpu.VMEM((RPT,), jnp.float32),
            pltpu.SemaphoreType.DMA,
        ],
    )
    def k(dst_hbm, out_hbm, acc_sp, idx_v, ones_v, zero_v, ssem):
        c = lax.axis_index("c")
        s = lax.axis_index("s")
        w = c * NS + s
        sl = pl.ds(s * RPT, RPT)

        @pl.loop(0, CHUNK, step=16)
        def _(i):
            ones_v[pl.ds(i, 16)] = jnp.ones((16,), jnp.float32)

        @pl.loop(0, RPT, step=16)
        def _(i):
            zero_v[pl.ds(i, 16)] = jnp.zeros((16,), jnp.float32)

        pltpu.sync_copy(zero_v, acc_sp.at[sl])
        pltpu.sync_copy(dst_hbm.at[pl.ds(w * KW, KW)], idx_v.at[pl.ds(0, KW)])

        @pl.when(w < XW)
        def _():
            pltpu.sync_copy(dst_hbm.at[pl.ds(NW * KW + w, 1)],
                            idx_v.at[pl.ds(KW, 1)])

        plsc.subcore_barrier()

        @pl.loop(0, KW)
        def _(j):
            pltpu.async_copy(ones_v, acc_sp.at[idx_v.at[j]], ssem, add=True)

        @pl.when(w < XW)
        def _():
            pltpu.async_copy(ones_v, acc_sp.at[idx_v.at[KW]], ssem, add=True)

        @pl.loop(0, KW)
        def _(j):
            pltpu.make_async_copy(ones_v, acc_sp.at[idx_v.at[0]], ssem).wait()

        @pl.when(w < XW)
        def _():
            pltpu.make_async_copy(ones_v, acc_sp.at[idx_v.at[0]], ssem).wait()

        plsc.subcore_barrier()
        pltpu.sync_copy(acc_sp.at[sl], out_hbm.at[c].at[sl])

    return k(dst2)


def _sc_aggregate(table, src2, dst2):
    @functools.partial(
        pl.kernel,
        out_type=jax.ShapeDtypeStruct((NC, NP, H), jnp.float32),
        mesh=_mesh,
        compiler_params=_sc_params,
        scratch_types=[
            pltpu.VMEM_SHARED((NP, H), jnp.float32),
            pltpu.VMEM((KW + 1, CHUNK), jnp.int32),
            pltpu.VMEM((KW + 1, CHUNK), jnp.int32),
            pltpu.VMEM((NBUF * CHUNK, H), jnp.float32),
            pltpu.SemaphoreType.DMA((NBUF,)),
            pltpu.SemaphoreType.DMA((NBUF,)),
        ],
    )
    def k(table_hbm, src_hbm, dst_hbm, out_hbm, acc_sp, src_v, dst_v,
          rows_v, gsem, ssem):
        c = lax.axis_index("c")
        s = lax.axis_index("s")
        w = c * NS + s
        sl = pl.ds(s * RPT, RPT)
        bufs = [rows_v.at[pl.ds(b * CHUNK, CHUNK)] for b in range(NBUF)]

        @pl.loop(0, CHUNK)
        def _(i):
            rows_v[i, :] = jnp.zeros((16,), jnp.float32)

        @pl.loop(0, RPT // CHUNK)
        def _(r):
            pltpu.sync_copy(
                bufs[0], acc_sp.at[pl.ds(s * RPT + r * CHUNK, CHUNK)])

        pltpu.sync_copy(src_hbm.at[pl.ds(w * KW, KW)], src_v.at[pl.ds(0, KW)])
        pltpu.sync_copy(dst_hbm.at[pl.ds(w * KW, KW)], dst_v.at[pl.ds(0, KW)])

        @pl.when(w < XW)
        def _():
            pltpu.sync_copy(src_hbm.at[pl.ds(NW * KW + w, 1)],
                            src_v.at[pl.ds(KW, 1)])
            pltpu.sync_copy(dst_hbm.at[pl.ds(NW * KW + w, 1)],
                            dst_v.at[pl.ds(KW, 1)])

        plsc.subcore_barrier()

        def wait_gather(b):
            pltpu.make_async_copy(
                table_hbm.at[src_v.at[0]], bufs[b], gsem.at[b]).wait()

        def wait_scatter(b):
            pltpu.make_async_copy(
                bufs[b], acc_sp.at[dst_v.at[0]], ssem.at[b]).wait()

        for b in range(NBUF):
            pltpu.async_copy(table_hbm.at[src_v.at[b]], bufs[b], gsem.at[b])
        for b in range(NBUF):
            wait_gather(b)
            pltpu.async_copy(bufs[b], acc_sp.at[dst_v.at[b]], ssem.at[b],
                             add=True)

        @pl.loop(1, KW // NBUF)
        def _(r):
            base = r * NBUF
            for b in range(NBUF):
                wait_scatter(b)
                pltpu.async_copy(table_hbm.at[src_v.at[base + b]], bufs[b],
                                 gsem.at[b])
            for b in range(NBUF):
                wait_gather(b)
                pltpu.async_copy(bufs[b], acc_sp.at[dst_v.at[base + b]],
                                 ssem.at[b], add=True)

        for b in range(NBUF):
            wait_scatter(b)

        @pl.when(w < XW)
        def _():
            pltpu.sync_copy(table_hbm.at[src_v.at[KW]], bufs[0])
            pltpu.sync_copy(bufs[0], acc_sp.at[dst_v.at[KW]], add=True)

        plsc.subcore_barrier()
        pltpu.sync_copy(acc_sp.at[sl], out_hbm.at[c].at[sl])

    return k(table, src2, dst2)


def _tc_matmul(x, W1):
    def body(x_ref, w_ref, o_ref):
        o_ref[...] = jnp.dot(x_ref[...], w_ref[...],
                             preferred_element_type=jnp.float32)

    return pl.pallas_call(
        body,
        out_shape=jax.ShapeDtypeStruct((N, H), jnp.float32),
    )(x, W1)


def _dinv_from_parts(degp_t_ref):
    deg = degp_t_ref[:, 0:1] + degp_t_ref[:, 1:2] + 1.0
    return lax.rsqrt(deg)


def _tc_prescale(h, degp_t):
    def body(h_ref, d_ref, o_ref):
        dinv = _dinv_from_parts(d_ref)
        o_ref[0:N, :] = dinv[0:N] * h_ref[...]
        o_ref[N:NP, :] = jnp.zeros((NP - N, H), jnp.float32)

    return pl.pallas_call(
        body,
        out_shape=jax.ShapeDtypeStruct((NP, H), jnp.float32),
    )(h, degp_t)


def _tc_mid(agg1, hp, degp_t, b1, W2p):
    def body(a_ref, hp_ref, d_ref, b1_ref, w2_ref, o_ref):
        dinv = _dinv_from_parts(d_ref)
        pre = dinv * (a_ref[0] + a_ref[1] + hp_ref[...]) + b1_ref[...]
        out1 = jnp.maximum(pre, 0.0)
        h2 = jnp.dot(out1, w2_ref[...], preferred_element_type=jnp.float32)
        o_ref[0:N, :] = dinv[0:N] * h2[0:N, :]
        o_ref[N:NP, :] = jnp.zeros((NP - N, H), jnp.float32)

    return pl.pallas_call(
        body,
        out_shape=jax.ShapeDtypeStruct((NP, H), jnp.float32),
    )(agg1, hp, degp_t, b1, W2p)


def _tc_final(agg2, hp2, degp_t, b2):
    def body(a_ref, hp_ref, d_ref, b2_ref, o_ref):
        dinv = _dinv_from_parts(d_ref)
        z16 = dinv * (a_ref[0] + a_ref[1] + hp_ref[...])
        z = z16[0:N, 0:C] + b2_ref[...]
        m = jnp.max(z, axis=1, keepdims=True)
        zm = z - m
        lse = jnp.log(jnp.sum(jnp.exp(zm), axis=1, keepdims=True))
        o_ref[...] = zm - lse

    return pl.pallas_call(
        body,
        out_shape=jax.ShapeDtypeStruct((N, C), jnp.float32),
    )(agg2, hp2, degp_t, b2)


def kernel(x, edge_index, W1, b1, W2, b2):
    src2 = edge_index[0].reshape(ER, CHUNK)
    dst2 = edge_index[1].reshape(ER, CHUNK)

    degp = _sc_degree(dst2)
    h = _tc_matmul(x, W1)
    degp_t = degp.T

    hp = _tc_prescale(h, degp_t)
    agg1 = _sc_aggregate(hp, src2, dst2)

    b1r = b1.reshape(1, H)
    W2p = jnp.concatenate([W2, jnp.zeros((H, H - C), jnp.float32)], axis=1)
    hp2 = _tc_mid(agg1, hp, degp_t, b1r, W2p)
    agg2 = _sc_aggregate(hp2, src2, dst2)

    return _tc_final(agg2, hp2, degp_t, b2.reshape(1, C))


# speedup vs baseline: 1.1244x; 1.1244x over previous
import functools

import jax
import jax.numpy as jnp
from jax import lax
from jax.experimental import pallas as pl
from jax.experimental.pallas import tpu as pltpu
from jax.experimental.pallas import tpu_sc as plsc

N = 10000
E = 320000
D = 128
H = 16
C = 7

NC = 2
NS = 16
NW = NC * NS
CHUNK = 128
ER = E // CHUNK
KW = 78
XW = ER - NW * KW
NBUF = 13
NP = 10240
RPT = NP // NS

_mesh = plsc.VectorSubcoreMesh(core_axis_name="c", subcore_axis_name="s")
_sc_params = pltpu.CompilerParams(use_tc_tiling_on_sc=False)


def _sc_degree(ei2):
    @functools.partial(
        pl.kernel,
        out_type=jax.ShapeDtypeStruct((NC, NP), jnp.float32),
        mesh=_mesh,
        compiler_params=_sc_params,
        scratch_types=[
            pltpu.VMEM_SHARED((NP,), jnp.float32),
            pltpu.VMEM((KW + 1, CHUNK), jnp.int32),
            pltpu.VMEM((CHUNK,), jnp.float32),
            pltpu.VMEM((RPT,), jnp.float32),
            pltpu.SemaphoreType.DMA,
        ],
    )
    def k(ei_hbm, out_hbm, acc_sp, idx_v, ones_v, zero_v, ssem):
        c = lax.axis_index("c")
        s = lax.axis_index("s")
        w = c * NS + s
        sl = pl.ds(s * RPT, RPT)
        dst_hbm = ei_hbm.at[1]

        @pl.loop(0, CHUNK, step=16)
        def _(i):
            ones_v[pl.ds(i, 16)] = jnp.ones((16,), jnp.float32)

        @pl.loop(0, RPT, step=16)
        def _(i):
            zero_v[pl.ds(i, 16)] = jnp.zeros((16,), jnp.float32)

        pltpu.sync_copy(zero_v, acc_sp.at[sl])
        pltpu.sync_copy(dst_hbm.at[pl.ds(w * KW, KW)], idx_v.at[pl.ds(0, KW)])

        @pl.when(w < XW)
        def _():
            pltpu.sync_copy(dst_hbm.at[pl.ds(NW * KW + w, 1)],
                            idx_v.at[pl.ds(KW, 1)])

        plsc.subcore_barrier()

        @pl.loop(0, KW)
        def _(j):
            pltpu.async_copy(ones_v, acc_sp.at[idx_v.at[j]], ssem, add=True)

        @pl.when(w < XW)
        def _():
            pltpu.async_copy(ones_v, acc_sp.at[idx_v.at[KW]], ssem, add=True)

        @pl.loop(0, KW)
        def _(j):
            pltpu.make_async_copy(ones_v, acc_sp.at[idx_v.at[0]], ssem).wait()

        @pl.when(w < XW)
        def _():
            pltpu.make_async_copy(ones_v, acc_sp.at[idx_v.at[0]], ssem).wait()

        plsc.subcore_barrier()
        pltpu.sync_copy(acc_sp.at[sl], out_hbm.at[c].at[sl])

    return k(ei2)


def _sc_aggregate(table, ei2):
    @functools.partial(
        pl.kernel,
        out_type=jax.ShapeDtypeStruct((NC, NP, H), jnp.float32),
        mesh=_mesh,
        compiler_params=_sc_params,
        scratch_types=[
            pltpu.VMEM_SHARED((NP, H), jnp.float32),
            pltpu.VMEM((KW + 1, CHUNK), jnp.int32),
            pltpu.VMEM((KW + 1, CHUNK), jnp.int32),
            pltpu.VMEM((NBUF * CHUNK, H), jnp.float32),
            pltpu.SemaphoreType.DMA((NBUF,)),
            pltpu.SemaphoreType.DMA((NBUF,)),
        ],
    )
    def k(table_hbm, ei_hbm, out_hbm, acc_sp, src_v, dst_v,
          rows_v, gsem, ssem):
        c = lax.axis_index("c")
        s = lax.axis_index("s")
        w = c * NS + s
        sl = pl.ds(s * RPT, RPT)
        src_hbm = ei_hbm.at[0]
        dst_hbm = ei_hbm.at[1]
        bufs = [rows_v.at[pl.ds(b * CHUNK, CHUNK)] for b in range(NBUF)]

        @pl.loop(0, CHUNK)
        def _(i):
            rows_v[i, :] = jnp.zeros((16,), jnp.float32)

        @pl.loop(0, RPT // CHUNK)
        def _(r):
            pltpu.sync_copy(
                bufs[0], acc_sp.at[pl.ds(s * RPT + r * CHUNK, CHUNK)])

        pltpu.sync_copy(src_hbm.at[pl.ds(w * KW, KW)], src_v.at[pl.ds(0, KW)])
        pltpu.sync_copy(dst_hbm.at[pl.ds(w * KW, KW)], dst_v.at[pl.ds(0, KW)])

        @pl.when(w < XW)
        def _():
            pltpu.sync_copy(src_hbm.at[pl.ds(NW * KW + w, 1)],
                            src_v.at[pl.ds(KW, 1)])
            pltpu.sync_copy(dst_hbm.at[pl.ds(NW * KW + w, 1)],
                            dst_v.at[pl.ds(KW, 1)])

        plsc.subcore_barrier()

        def wait_gather(b):
            pltpu.make_async_copy(
                table_hbm.at[src_v.at[0]], bufs[b], gsem.at[b]).wait()

        def wait_scatter(b):
            pltpu.make_async_copy(
                bufs[b], acc_sp.at[dst_v.at[0]], ssem.at[b]).wait()

        for b in range(NBUF):
            pltpu.async_copy(table_hbm.at[src_v.at[b]], bufs[b], gsem.at[b])
        for b in range(NBUF):
            wait_gather(b)
            pltpu.async_copy(bufs[b], acc_sp.at[dst_v.at[b]], ssem.at[b],
                             add=True)

        @pl.loop(1, KW // NBUF)
        def _(r):
            base = r * NBUF
            for b in range(NBUF):
                wait_scatter(b)
                pltpu.async_copy(table_hbm.at[src_v.at[base + b]], bufs[b],
                                 gsem.at[b])
            for b in range(NBUF):
                wait_gather(b)
                pltpu.async_copy(bufs[b], acc_sp.at[dst_v.at[base + b]],
                                 ssem.at[b], add=True)

        for b in range(NBUF):
            wait_scatter(b)

        @pl.when(w < XW)
        def _():
            pltpu.sync_copy(table_hbm.at[src_v.at[KW]], bufs[0])
            pltpu.sync_copy(bufs[0], acc_sp.at[dst_v.at[KW]], add=True)

        plsc.subcore_barrier()
        pltpu.sync_copy(acc_sp.at[sl], out_hbm.at[c].at[sl])

    return k(table, ei2)


def _tc_matmul(x, W1):
    def body(x_ref, w_ref, o_ref):
        o_ref[...] = jnp.dot(x_ref[...], w_ref[...],
                             preferred_element_type=jnp.float32)

    return pl.pallas_call(
        body,
        out_shape=jax.ShapeDtypeStruct((N, H), jnp.float32),
    )(x, W1)


def _dinv_from_parts(degp_t_ref):
    deg = degp_t_ref[:, 0:1] + degp_t_ref[:, 1:2] + 1.0
    return lax.rsqrt(deg)


def _tc_prescale(h, degp_t):
    def body(h_ref, d_ref, o_ref):
        dinv = _dinv_from_parts(d_ref)
        o_ref[0:N, :] = dinv[0:N] * h_ref[...]
        o_ref[N:NP, :] = jnp.zeros((NP - N, H), jnp.float32)

    return pl.pallas_call(
        body,
        out_shape=jax.ShapeDtypeStruct((NP, H), jnp.float32),
    )(h, degp_t)


def _tc_mid(agg1, hp, degp_t, b1, W2p):
    def body(a_ref, hp_ref, d_ref, b1_ref, w2_ref, o_ref):
        dinv = _dinv_from_parts(d_ref)
        pre = dinv * (a_ref[0] + a_ref[1] + hp_ref[...]) + b1_ref[...]
        out1 = jnp.maximum(pre, 0.0)
        h2 = jnp.dot(out1, w2_ref[...], preferred_element_type=jnp.float32)
        o_ref[0:N, :] = dinv[0:N] * h2[0:N, :]
        o_ref[N:NP, :] = jnp.zeros((NP - N, H), jnp.float32)

    return pl.pallas_call(
        body,
        out_shape=jax.ShapeDtypeStruct((NP, H), jnp.float32),
    )(agg1, hp, degp_t, b1, W2p)


def _tc_final(agg2, hp2, degp_t, b2):
    def body(a_ref, hp_ref, d_ref, b2_ref, o_ref):
        dinv = _dinv_from_parts(d_ref)
        z16 = dinv * (a_ref[0] + a_ref[1] + hp_ref[...])
        z = z16[0:N, 0:C] + b2_ref[...]
        m = jnp.max(z, axis=1, keepdims=True)
        zm = z - m
        lse = jnp.log(jnp.sum(jnp.exp(zm), axis=1, keepdims=True))
        o_ref[...] = zm - lse

    return pl.pallas_call(
        body,
        out_shape=jax.ShapeDtypeStruct((N, C), jnp.float32),
    )(agg2, hp2, degp_t, b2)


def kernel(x, edge_index, W1, b1, W2, b2):
    ei2 = edge_index.reshape(2, ER, CHUNK)

    degp = _sc_degree(ei2)
    h = _tc_matmul(x, W1)
    degp_t = degp.T

    hp = _tc_prescale(h, degp_t)
    agg1 = _sc_aggregate(hp, ei2)

    b1r = b1.reshape(1, H)
    W2p = jnp.concatenate([W2, jnp.zeros((H, H - C), jnp.float32)], axis=1)
    hp2 = _tc_mid(agg1, hp, degp_t, b1r, W2p)
    agg2 = _sc_aggregate(hp2, ei2)

    return _tc_final(agg2, hp2, degp_t, b2.reshape(1, C))
